# baseline (speedup 1.0000x reference)
import functools

import jax
import jax.numpy as jnp
from jax import lax
from jax.experimental import pallas as pl
from jax.experimental.pallas import tpu as pltpu
from jax.experimental.pallas import tpu_sc as plsc

NC = 2
NS = 16
L = 16
NW = NC * NS
K = 16


def _make_edge_pass(n_nodes, d, chunks_per_w):
    rows_per_tile = n_nodes // NS
    mesh = plsc.VectorSubcoreMesh(core_axis_name="c", subcore_axis_name="s")

    cpw = chunks_per_w

    @functools.partial(
        pl.kernel,
        mesh=mesh,
        compiler_params=pltpu.CompilerParams(needs_layout_passes=False),
        out_type=jax.ShapeDtypeStruct((NC, n_nodes, d), jnp.float32),
        scratch_types=[
            pltpu.VMEM((2, 3 * K), jnp.float32),
            pltpu.VMEM((2, K), jnp.int32),
            pltpu.VMEM((2, K), jnp.int32),
            pltpu.VMEM((K, d), jnp.float32),
            pltpu.VMEM((K, d), jnp.float32),
            pltpu.VMEM((K * L,), jnp.float32),
            pltpu.VMEM((K * L,), jnp.float32),
            pltpu.VMEM_SHARED((n_nodes // 2, d), jnp.float32),
            pltpu.VMEM_SHARED((n_nodes, d), jnp.float32),
            pltpu.SemaphoreType.DMA,
            pltpu.SemaphoreType.DMA,
            pltpu.SemaphoreType.DMA,
            pltpu.SemaphoreType.DMA,
        ],
    )
    def edge_pass(sup_hbm, esl_hbm, zeros_hbm, out_hbm,
                  esl_v, hsrc_v, dst_v, rows0_v, rows1_v, vA_v, vC_v,
                  tab_sh, acc_sh, isem0, isem1, gsem0, gsem1):
        c = lax.axis_index("c")
        s = lax.axis_index("s")
        wid = s * NC + c
        r0 = s * rows_per_tile
        rows_bufs = (rows0_v, rows1_v)
        isems = (isem0, isem1)
        gsems = (gsem0, gsem1)
        base = wid * cpw

        def issue_idx(cid, b):
            off = pl.ds((base + cid) * 3 * K, 3 * K)
            pltpu.async_copy(esl_hbm.at[off], esl_v.at[b], isems[b])

        def wait_idx(cid, b):
            off = pl.ds((base + cid) * 3 * K, 3 * K)
            pltpu.make_async_copy(esl_hbm.at[off], esl_v.at[b],
                                  isems[b]).wait()

        def prep_gather(b):
            for g in range(K // L):
                sl = pl.ds(g * L, L)
                sv = plsc.bitcast(esl_v[b, sl], jnp.int32)
                hsrc_v[b, sl] = lax.shift_right_logical(sv, 1)
                dv = plsc.bitcast(esl_v[b, pl.ds(K + g * L, L)], jnp.int32)
                dst_v[b, sl] = dv
            pltpu.async_copy(tab_sh.at[hsrc_v.at[b]], rows_bufs[b], gsems[b])

        issue_idx(0, 0)
        pltpu.sync_copy(zeros_hbm.at[pl.ds(r0, rows_per_tile)],
                        acc_sh.at[pl.ds(r0, rows_per_tile)])

        @pl.when(s < NS // 2)
        def _():
            slab = pl.ds(s * rows_per_tile, rows_per_tile)
            pltpu.sync_copy(sup_hbm.at[slab], tab_sh.at[slab])

        plsc.subcore_barrier()
        issue_idx(1, 1)
        wait_idx(0, 0)
        prep_gather(0)

        def pair_body(i, carry):
            for b in range(2):
                cid = 2 * i + b
                nb = 1 - b
                rows_b = rows_bufs[b]

                @pl.when(cid + 1 < cpw)
                def _():
                    wait_idx(cid + 1, nb)
                    prep_gather(nb)

                for g in range(K // L):
                    sl = pl.ds(g * L, L)
                    vv = esl_v[b, pl.ds(2 * K + g * L, L)]
                    sv = plsc.bitcast(esl_v[b, sl], jnp.int32)
                    pv = jnp.bitwise_and(sv, jnp.int32(1)).astype(jnp.float32)
                    cc = vv * pv
                    rowbase = (jnp.arange(L, dtype=jnp.int32) + g * L) * L
                    for j in range(L):
                        plsc.store_scatter(vA_v, [rowbase + j], vv)
                        plsc.store_scatter(vC_v, [rowbase + j], cc)

                pltpu.make_async_copy(tab_sh.at[hsrc_v.at[b]], rows_b,
                                      gsems[b]).wait()

                def scale_e(e, carry2):
                    a_sl = pl.ds(e * L, L)
                    A = vA_v[a_sl]
                    C = vC_v[a_sl]
                    for fg in range(d // L):
                        sl = pl.ds(fg * L, L)
                        u = plsc.bitcast(rows_b[e, sl], jnp.int32)
                        lo = plsc.bitcast(lax.shift_left(u, 16), jnp.float32)
                        hi = plsc.bitcast(
                            jnp.bitwise_and(u, jnp.int32(-65536)), jnp.float32)
                        rows_b[e, sl] = lo * A + (hi - lo) * C
                    return carry2

                lax.fori_loop(0, K, scale_e, 0)
                pltpu.sync_copy(rows_b, acc_sh.at[dst_v.at[b]], add=True)

                @pl.when(cid + 2 < cpw)
                def _():
                    issue_idx(cid + 2, b)
            return carry

        lax.fori_loop(0, cpw // 2, pair_body, 0)
        plsc.subcore_barrier()
        pltpu.sync_copy(acc_sh.at[pl.ds(r0, rows_per_tile)],
                        out_hbm.at[c, pl.ds(r0, rows_per_tile)])

    return edge_pass


def _mm_body(x_ref, w_ref, o_ref):
    o_ref[...] = jnp.dot(x_ref[...], w_ref[...],
                         preferred_element_type=jnp.float32
                         ).astype(jnp.bfloat16)


def _combine_relu_mm_body(p0_ref, p1_ref, b_ref, w_ref, o_ref):
    h = jnp.maximum(p0_ref[...] + p1_ref[...] + b_ref[...], 0.0)
    o_ref[...] = jnp.dot(h, w_ref[...], preferred_element_type=jnp.float32
                         ).astype(jnp.bfloat16)


def _combine_bias_body(p0_ref, p1_ref, b_ref, o_ref):
    o_ref[...] = p0_ref[...] + p1_ref[...] + b_ref[...]


def _matmul_bf16(x, w, bm):
    n, d = x.shape
    return pl.pallas_call(
        _mm_body,
        grid=(n // bm,),
        in_specs=[pl.BlockSpec((bm, d), lambda i: (i, 0)),
                  pl.BlockSpec((d, w.shape[1]), lambda i: (0, 0))],
        out_specs=pl.BlockSpec((bm, w.shape[1]), lambda i: (i, 0)),
        out_shape=jax.ShapeDtypeStruct((n, w.shape[1]), jnp.bfloat16),
    )(x, w)


def _combine_relu_matmul_bf16(p0, p1, b, w, bm):
    n, d = p0.shape
    return pl.pallas_call(
        _combine_relu_mm_body,
        grid=(n // bm,),
        in_specs=[pl.BlockSpec((bm, d), lambda i: (i, 0)),
                  pl.BlockSpec((bm, d), lambda i: (i, 0)),
                  pl.BlockSpec((1, d), lambda i: (0, 0)),
                  pl.BlockSpec((d, w.shape[1]), lambda i: (0, 0))],
        out_specs=pl.BlockSpec((bm, w.shape[1]), lambda i: (i, 0)),
        out_shape=jax.ShapeDtypeStruct((n, w.shape[1]), jnp.bfloat16),
    )(p0, p1, b.reshape(1, d), w)


def _combine_bias(p0, p1, b, bm):
    n, d = p0.shape
    return pl.pallas_call(
        _combine_bias_body,
        grid=(n // bm,),
        in_specs=[pl.BlockSpec((bm, d), lambda i: (i, 0)),
                  pl.BlockSpec((bm, d), lambda i: (i, 0)),
                  pl.BlockSpec((1, d), lambda i: (0, 0))],
        out_specs=pl.BlockSpec((bm, d), lambda i: (i, 0)),
        out_shape=jax.ShapeDtypeStruct((n, d), jnp.float32),
    )(p0, p1, b.reshape(1, d))


def kernel(features, edge_index, adj_values, W1, b1, W3, b3):
    n_nodes, d = features.shape
    n_edges = edge_index.shape[1]

    dst = edge_index[0].astype(jnp.int32)
    src = edge_index[1].astype(jnp.int32)
    val = adj_values.astype(jnp.float32)

    per_w = NW * K * 2
    e_pad = ((n_edges + per_w - 1) // per_w) * per_w
    chunks_per_w = e_pad // (NW * K)
    pad = e_pad - n_edges
    if pad:
        src = jnp.pad(src, (0, pad))
        dst = jnp.pad(dst, (0, pad))
        val = jnp.pad(val, (0, pad))

    n_chunks = e_pad // K
    esl = jnp.stack([
        lax.bitcast_convert_type(src, jnp.float32),
        lax.bitcast_convert_type(dst, jnp.float32),
        val,
    ]).reshape(3, n_chunks, K).transpose(1, 0, 2).reshape(-1)

    row_q = NS * 8
    n_pad = ((n_nodes + row_q - 1) // row_q) * row_q

    zeros = jnp.zeros((n_pad, d), jnp.float32)
    edge_pass = _make_edge_pass(n_pad, d, chunks_per_w)

    bm = 1000 if n_nodes % 1000 == 0 else n_nodes

    def pack_rows(x_bf):
        if n_pad != n_nodes:
            x_bf = jnp.concatenate(
                [x_bf, jnp.zeros((n_pad - n_nodes, d), x_bf.dtype)], axis=0)
        x3 = x_bf.reshape(n_pad // 2, 2, d).transpose(0, 2, 1)
        return lax.bitcast_convert_type(x3, jnp.float32)

    support1 = pack_rows(_matmul_bf16(features, W1, bm))
    p = edge_pass(support1, esl, zeros)
    support2 = pack_rows(
        _combine_relu_matmul_bf16(p[0, :n_nodes], p[1, :n_nodes], b1, W3, bm))
    q = edge_pass(support2, esl, zeros)
    return _combine_bias(q[0, :n_nodes], q[1, :n_nodes], b3, bm)


# speedup vs baseline: 1.1484x; 1.1484x over previous
import functools

import jax
import jax.numpy as jnp
from jax import lax
from jax.experimental import pallas as pl
from jax.experimental.pallas import tpu as pltpu
from jax.experimental.pallas import tpu_sc as plsc

NC = 2
NS = 16
L = 16
NW = NC * NS
K = 16


def _make_edge_pass(n_nodes, d, chunks_per_w):
    rows_per_tile = n_nodes // NS
    mesh = plsc.VectorSubcoreMesh(core_axis_name="c", subcore_axis_name="s")

    cpw = chunks_per_w

    @functools.partial(
        pl.kernel,
        mesh=mesh,
        compiler_params=pltpu.CompilerParams(needs_layout_passes=False),
        out_type=jax.ShapeDtypeStruct((NC, n_nodes, d), jnp.float32),
        scratch_types=[
            pltpu.VMEM((2, 3 * K), jnp.float32),
            pltpu.VMEM((2, K), jnp.int32),
            pltpu.VMEM((2, K), jnp.int32),
            pltpu.VMEM((K, d), jnp.float32),
            pltpu.VMEM((K, d), jnp.float32),
            pltpu.VMEM((K * L,), jnp.float32),
            pltpu.VMEM((K * L,), jnp.float32),
            pltpu.VMEM_SHARED((n_nodes // 2, d), jnp.float32),
            pltpu.VMEM_SHARED((n_nodes, d), jnp.float32),
            pltpu.SemaphoreType.DMA,
            pltpu.SemaphoreType.DMA,
            pltpu.SemaphoreType.DMA,
            pltpu.SemaphoreType.DMA,
            pltpu.SemaphoreType.DMA,
            pltpu.SemaphoreType.DMA,
        ],
    )
    def edge_pass(sup_hbm, esl_hbm, zeros_hbm, out_hbm,
                  esl_v, hsrc_v, dst_v, rows0_v, rows1_v, vA_v, vC_v,
                  tab_sh, acc_sh, isem0, isem1, gsem0, gsem1, ssem0, ssem1):
        c = lax.axis_index("c")
        s = lax.axis_index("s")
        wid = s * NC + c
        r0 = s * rows_per_tile
        rows_bufs = (rows0_v, rows1_v)
        isems = (isem0, isem1)
        gsems = (gsem0, gsem1)
        ssems = (ssem0, ssem1)
        base = wid * cpw

        def issue_idx(cid, b):
            off = pl.ds((base + cid) * 3 * K, 3 * K)
            pltpu.async_copy(esl_hbm.at[off], esl_v.at[b], isems[b])

        def wait_idx(cid, b):
            off = pl.ds((base + cid) * 3 * K, 3 * K)
            pltpu.make_async_copy(esl_hbm.at[off], esl_v.at[b],
                                  isems[b]).wait()

        def prep_gather(b):
            for g in range(K // L):
                sl = pl.ds(g * L, L)
                sv = plsc.bitcast(esl_v[b, sl], jnp.int32)
                hsrc_v[b, sl] = lax.shift_right_logical(sv, 1)
                dv = plsc.bitcast(esl_v[b, pl.ds(K + g * L, L)], jnp.int32)
                dst_v[b, sl] = dv
            pltpu.async_copy(tab_sh.at[hsrc_v.at[b]], rows_bufs[b], gsems[b])

        issue_idx(0, 0)
        pltpu.sync_copy(zeros_hbm.at[pl.ds(r0, rows_per_tile)],
                        acc_sh.at[pl.ds(r0, rows_per_tile)])

        @pl.when(s < NS // 2)
        def _():
            slab = pl.ds(s * rows_per_tile, rows_per_tile)
            pltpu.sync_copy(sup_hbm.at[slab], tab_sh.at[slab])

        plsc.subcore_barrier()
        issue_idx(1, 1)
        wait_idx(0, 0)
        prep_gather(0)

        def pair_body(i, carry):
            for b in range(2):
                cid = 2 * i + b
                nb = 1 - b
                rows_b = rows_bufs[b]

                @pl.when(cid >= 1)
                def _():
                    pltpu.make_async_copy(
                        rows_bufs[nb], acc_sh.at[dst_v.at[nb]],
                        ssems[nb]).wait()

                @pl.when(cid + 1 < cpw)
                def _():
                    wait_idx(cid + 1, nb)
                    prep_gather(nb)

                for g in range(K // L):
                    sl = pl.ds(g * L, L)
                    vv = esl_v[b, pl.ds(2 * K + g * L, L)]
                    sv = plsc.bitcast(esl_v[b, sl], jnp.int32)
                    pv = jnp.bitwise_and(sv, jnp.int32(1)).astype(jnp.float32)
                    cc = vv * pv
                    rowbase = (jnp.arange(L, dtype=jnp.int32) + g * L) * L
                    for j in range(L):
                        plsc.store_scatter(vA_v, [rowbase + j], vv)
                        plsc.store_scatter(vC_v, [rowbase + j], cc)

                pltpu.make_async_copy(tab_sh.at[hsrc_v.at[b]], rows_b,
                                      gsems[b]).wait()

                def scale_e(e, carry2):
                    a_sl = pl.ds(e * L, L)
                    A = vA_v[a_sl]
                    C = vC_v[a_sl]
                    for fg in range(d // L):
                        sl = pl.ds(fg * L, L)
                        u = plsc.bitcast(rows_b[e, sl], jnp.int32)
                        lo = plsc.bitcast(lax.shift_left(u, 16), jnp.float32)
                        hi = plsc.bitcast(
                            jnp.bitwise_and(u, jnp.int32(-65536)), jnp.float32)
                        rows_b[e, sl] = lo * A + (hi - lo) * C
                    return carry2

                lax.fori_loop(0, K, scale_e, 0)
                pltpu.async_copy(rows_b, acc_sh.at[dst_v.at[b]], ssems[b],
                                 add=True)

                @pl.when(cid + 2 < cpw)
                def _():
                    issue_idx(cid + 2, b)
            return carry

        lax.fori_loop(0, cpw // 2, pair_body, 0)
        pltpu.make_async_copy(rows_bufs[1], acc_sh.at[dst_v.at[1]],
                              ssems[1]).wait()
        plsc.subcore_barrier()
        pltpu.sync_copy(acc_sh.at[pl.ds(r0, rows_per_tile)],
                        out_hbm.at[c, pl.ds(r0, rows_per_tile)])

    return edge_pass


def _mm_body(x_ref, w_ref, o_ref):
    o_ref[...] = jnp.dot(x_ref[...], w_ref[...],
                         preferred_element_type=jnp.float32
                         ).astype(jnp.bfloat16)


def _combine_relu_mm_body(p0_ref, p1_ref, b_ref, w_ref, o_ref):
    h = jnp.maximum(p0_ref[...] + p1_ref[...] + b_ref[...], 0.0)
    o_ref[...] = jnp.dot(h, w_ref[...], preferred_element_type=jnp.float32
                         ).astype(jnp.bfloat16)


def _combine_bias_body(p0_ref, p1_ref, b_ref, o_ref):
    o_ref[...] = p0_ref[...] + p1_ref[...] + b_ref[...]


def _matmul_bf16(x, w, bm):
    n, d = x.shape
    return pl.pallas_call(
        _mm_body,
        grid=(n // bm,),
        in_specs=[pl.BlockSpec((bm, d), lambda i: (i, 0)),
                  pl.BlockSpec((d, w.shape[1]), lambda i: (0, 0))],
        out_specs=pl.BlockSpec((bm, w.shape[1]), lambda i: (i, 0)),
        out_shape=jax.ShapeDtypeStruct((n, w.shape[1]), jnp.bfloat16),
    )(x, w)


def _combine_relu_matmul_bf16(p0, p1, b, w, bm):
    n, d = p0.shape
    return pl.pallas_call(
        _combine_relu_mm_body,
        grid=(n // bm,),
        in_specs=[pl.BlockSpec((bm, d), lambda i: (i, 0)),
                  pl.BlockSpec((bm, d), lambda i: (i, 0)),
                  pl.BlockSpec((1, d), lambda i: (0, 0)),
                  pl.BlockSpec((d, w.shape[1]), lambda i: (0, 0))],
        out_specs=pl.BlockSpec((bm, w.shape[1]), lambda i: (i, 0)),
        out_shape=jax.ShapeDtypeStruct((n, w.shape[1]), jnp.bfloat16),
    )(p0, p1, b.reshape(1, d), w)


def _combine_bias(p0, p1, b, bm):
    n, d = p0.shape
    return pl.pallas_call(
        _combine_bias_body,
        grid=(n // bm,),
        in_specs=[pl.BlockSpec((bm, d), lambda i: (i, 0)),
                  pl.BlockSpec((bm, d), lambda i: (i, 0)),
                  pl.BlockSpec((1, d), lambda i: (0, 0))],
        out_specs=pl.BlockSpec((bm, d), lambda i: (i, 0)),
        out_shape=jax.ShapeDtypeStruct((n, d), jnp.float32),
    )(p0, p1, b.reshape(1, d))


def kernel(features, edge_index, adj_values, W1, b1, W3, b3):
    n_nodes, d = features.shape
    n_edges = edge_index.shape[1]

    dst = edge_index[0].astype(jnp.int32)
    src = edge_index[1].astype(jnp.int32)
    val = adj_values.astype(jnp.float32)

    per_w = NW * K * 2
    e_pad = ((n_edges + per_w - 1) // per_w) * per_w
    chunks_per_w = e_pad // (NW * K)
    pad = e_pad - n_edges
    if pad:
        src = jnp.pad(src, (0, pad))
        dst = jnp.pad(dst, (0, pad))
        val = jnp.pad(val, (0, pad))

    n_chunks = e_pad // K
    esl = jnp.stack([
        lax.bitcast_convert_type(src, jnp.float32),
        lax.bitcast_convert_type(dst, jnp.float32),
        val,
    ]).reshape(3, n_chunks, K).transpose(1, 0, 2).reshape(-1)

    row_q = NS * 8
    n_pad = ((n_nodes + row_q - 1) // row_q) * row_q

    zeros = jnp.zeros((n_pad, d), jnp.float32)
    edge_pass = _make_edge_pass(n_pad, d, chunks_per_w)

    bm = 1000 if n_nodes % 1000 == 0 else n_nodes

    def pack_rows(x_bf):
        if n_pad != n_nodes:
            x_bf = jnp.concatenate(
                [x_bf, jnp.zeros((n_pad - n_nodes, d), x_bf.dtype)], axis=0)
        x3 = x_bf.reshape(n_pad // 2, 2, d).transpose(0, 2, 1)
        return lax.bitcast_convert_type(x3, jnp.float32)

    support1 = pack_rows(_matmul_bf16(features, W1, bm))
    p = edge_pass(support1, esl, zeros)
    support2 = pack_rows(
        _combine_relu_matmul_bf16(p[0, :n_nodes], p[1, :n_nodes], b1, W3, bm))
    q = edge_pass(support2, esl, zeros)
    return _combine_bias(q[0, :n_nodes], q[1, :n_nodes], b3, bm)


# speedup vs baseline: 1.1923x; 1.0382x over previous
import functools

import jax
import jax.numpy as jnp
from jax import lax
from jax.experimental import pallas as pl
from jax.experimental.pallas import tpu as pltpu
from jax.experimental.pallas import tpu_sc as plsc

NC = 2
NS = 16
L = 16
NW = NC * NS
K = 16


def _make_edge_pass(n_nodes, d, chunks_per_w):
    rows_per_tile = n_nodes // NS
    mesh = plsc.VectorSubcoreMesh(core_axis_name="c", subcore_axis_name="s")

    cpw = chunks_per_w

    @functools.partial(
        pl.kernel,
        mesh=mesh,
        compiler_params=pltpu.CompilerParams(needs_layout_passes=False),
        out_type=jax.ShapeDtypeStruct((NC, n_nodes, d), jnp.float32),
        scratch_types=[
            pltpu.VMEM((2, 3 * K), jnp.float32),
            pltpu.VMEM((2, K), jnp.int32),
            pltpu.VMEM((2, K), jnp.int32),
            pltpu.VMEM((K, d), jnp.float32),
            pltpu.VMEM((K, d), jnp.float32),
            pltpu.VMEM((K * L,), jnp.float32),
            pltpu.VMEM((K * L,), jnp.float32),
            pltpu.VMEM_SHARED((n_nodes // 2, d), jnp.float32),
            pltpu.VMEM_SHARED((n_nodes, d), jnp.float32),
            pltpu.SemaphoreType.DMA,
            pltpu.SemaphoreType.DMA,
            pltpu.SemaphoreType.DMA,
            pltpu.SemaphoreType.DMA,
            pltpu.SemaphoreType.DMA,
            pltpu.SemaphoreType.DMA,
        ],
    )
    def edge_pass(sup_hbm, esl_hbm, zeros_hbm, out_hbm,
                  esl_v, hsrc_v, dst_v, rows0_v, rows1_v, vA_v, vC_v,
                  tab_sh, acc_sh, isem0, isem1, gsem0, gsem1, ssem0, ssem1):
        c = lax.axis_index("c")
        s = lax.axis_index("s")
        wid = s * NC + c
        r0 = s * rows_per_tile
        rows_bufs = (rows0_v, rows1_v)
        isems = (isem0, isem1)
        gsems = (gsem0, gsem1)
        ssems = (ssem0, ssem1)
        base = wid * cpw

        def issue_idx(cid, b):
            off = pl.ds((base + cid) * 3 * K, 3 * K)
            pltpu.async_copy(esl_hbm.at[off], esl_v.at[b], isems[b])

        def wait_idx(cid, b):
            off = pl.ds((base + cid) * 3 * K, 3 * K)
            pltpu.make_async_copy(esl_hbm.at[off], esl_v.at[b],
                                  isems[b]).wait()

        def prep_gather(b):
            for g in range(K // L):
                sl = pl.ds(g * L, L)
                sv = plsc.bitcast(esl_v[b, sl], jnp.int32)
                hsrc_v[b, sl] = lax.shift_right_logical(sv, 1)
                dv = plsc.bitcast(esl_v[b, pl.ds(K + g * L, L)], jnp.int32)
                dst_v[b, sl] = dv
            pltpu.async_copy(tab_sh.at[hsrc_v.at[b]], rows_bufs[b], gsems[b])

        issue_idx(0, 0)
        pltpu.sync_copy(zeros_hbm.at[pl.ds(r0, rows_per_tile)],
                        acc_sh.at[pl.ds(r0, rows_per_tile)])

        @pl.when(s < NS // 2)
        def _():
            slab = pl.ds(s * rows_per_tile, rows_per_tile)
            pltpu.sync_copy(sup_hbm.at[slab], tab_sh.at[slab])

        plsc.subcore_barrier()
        issue_idx(1, 1)
        wait_idx(0, 0)
        prep_gather(0)

        def pair_body(i, carry):
            for b in range(2):
                cid = 2 * i + b
                nb = 1 - b
                rows_b = rows_bufs[b]

                @pl.when(cid >= 1)
                def _():
                    pltpu.make_async_copy(
                        rows_bufs[nb], acc_sh.at[dst_v.at[nb]],
                        ssems[nb]).wait()

                @pl.when(cid + 1 < cpw)
                def _():
                    wait_idx(cid + 1, nb)
                    prep_gather(nb)

                for g in range(K // L):
                    sl = pl.ds(g * L, L)
                    vv = esl_v[b, pl.ds(2 * K + g * L, L)]
                    sv = plsc.bitcast(esl_v[b, sl], jnp.int32)
                    pv = jnp.bitwise_and(sv, jnp.int32(1)).astype(jnp.float32)
                    cc = vv * pv
                    rowbase = (jnp.arange(L, dtype=jnp.int32) + g * L) * L
                    for j in range(L):
                        plsc.store_scatter(vA_v, [rowbase + j], vv)
                        plsc.store_scatter(vC_v, [rowbase + j], cc)

                pltpu.make_async_copy(tab_sh.at[hsrc_v.at[b]], rows_b,
                                      gsems[b]).wait()

                for e in range(K):
                    a_sl = pl.ds(e * L, L)
                    A = vA_v[a_sl]
                    C = vC_v[a_sl]
                    for fg in range(d // L):
                        sl = pl.ds(fg * L, L)
                        u = plsc.bitcast(rows_b[e, sl], jnp.int32)
                        lo = plsc.bitcast(lax.shift_left(u, 16), jnp.float32)
                        hi = plsc.bitcast(
                            jnp.bitwise_and(u, jnp.int32(-65536)), jnp.float32)
                        rows_b[e, sl] = lo * A + (hi - lo) * C
                pltpu.async_copy(rows_b, acc_sh.at[dst_v.at[b]], ssems[b],
                                 add=True)

                @pl.when(cid + 2 < cpw)
                def _():
                    issue_idx(cid + 2, b)
            return carry

        lax.fori_loop(0, cpw // 2, pair_body, 0)
        pltpu.make_async_copy(rows_bufs[1], acc_sh.at[dst_v.at[1]],
                              ssems[1]).wait()
        plsc.subcore_barrier()
        pltpu.sync_copy(acc_sh.at[pl.ds(r0, rows_per_tile)],
                        out_hbm.at[c, pl.ds(r0, rows_per_tile)])

    return edge_pass


def _mm_body(x_ref, w_ref, o_ref):
    o_ref[...] = jnp.dot(x_ref[...], w_ref[...],
                         preferred_element_type=jnp.float32
                         ).astype(jnp.bfloat16)


def _combine_relu_mm_body(p0_ref, p1_ref, b_ref, w_ref, o_ref):
    h = jnp.maximum(p0_ref[...] + p1_ref[...] + b_ref[...], 0.0)
    o_ref[...] = jnp.dot(h, w_ref[...], preferred_element_type=jnp.float32
                         ).astype(jnp.bfloat16)


def _combine_bias_body(p0_ref, p1_ref, b_ref, o_ref):
    o_ref[...] = p0_ref[...] + p1_ref[...] + b_ref[...]


def _matmul_bf16(x, w, bm):
    n, d = x.shape
    return pl.pallas_call(
        _mm_body,
        grid=(n // bm,),
        in_specs=[pl.BlockSpec((bm, d), lambda i: (i, 0)),
                  pl.BlockSpec((d, w.shape[1]), lambda i: (0, 0))],
        out_specs=pl.BlockSpec((bm, w.shape[1]), lambda i: (i, 0)),
        out_shape=jax.ShapeDtypeStruct((n, w.shape[1]), jnp.bfloat16),
    )(x, w)


def _combine_relu_matmul_bf16(p0, p1, b, w, bm):
    n, d = p0.shape
    return pl.pallas_call(
        _combine_relu_mm_body,
        grid=(n // bm,),
        in_specs=[pl.BlockSpec((bm, d), lambda i: (i, 0)),
                  pl.BlockSpec((bm, d), lambda i: (i, 0)),
                  pl.BlockSpec((1, d), lambda i: (0, 0)),
                  pl.BlockSpec((d, w.shape[1]), lambda i: (0, 0))],
        out_specs=pl.BlockSpec((bm, w.shape[1]), lambda i: (i, 0)),
        out_shape=jax.ShapeDtypeStruct((n, w.shape[1]), jnp.bfloat16),
    )(p0, p1, b.reshape(1, d), w)


def _combine_bias(p0, p1, b, bm):
    n, d = p0.shape
    return pl.pallas_call(
        _combine_bias_body,
        grid=(n // bm,),
        in_specs=[pl.BlockSpec((bm, d), lambda i: (i, 0)),
                  pl.BlockSpec((bm, d), lambda i: (i, 0)),
                  pl.BlockSpec((1, d), lambda i: (0, 0))],
        out_specs=pl.BlockSpec((bm, d), lambda i: (i, 0)),
        out_shape=jax.ShapeDtypeStruct((n, d), jnp.float32),
    )(p0, p1, b.reshape(1, d))


def kernel(features, edge_index, adj_values, W1, b1, W3, b3):
    n_nodes, d = features.shape
    n_edges = edge_index.shape[1]

    dst = edge_index[0].astype(jnp.int32)
    src = edge_index[1].astype(jnp.int32)
    val = adj_values.astype(jnp.float32)

    per_w = NW * K * 2
    e_pad = ((n_edges + per_w - 1) // per_w) * per_w
    chunks_per_w = e_pad // (NW * K)
    pad = e_pad - n_edges
    if pad:
        src = jnp.pad(src, (0, pad))
        dst = jnp.pad(dst, (0, pad))
        val = jnp.pad(val, (0, pad))

    n_chunks = e_pad // K
    esl = jnp.stack([
        lax.bitcast_convert_type(src, jnp.float32),
        lax.bitcast_convert_type(dst, jnp.float32),
        val,
    ]).reshape(3, n_chunks, K).transpose(1, 0, 2).reshape(-1)

    row_q = NS * 8
    n_pad = ((n_nodes + row_q - 1) // row_q) * row_q

    zeros = jnp.zeros((n_pad, d), jnp.float32)
    edge_pass = _make_edge_pass(n_pad, d, chunks_per_w)

    bm = 1000 if n_nodes % 1000 == 0 else n_nodes

    def pack_rows(x_bf):
        if n_pad != n_nodes:
            x_bf = jnp.concatenate(
                [x_bf, jnp.zeros((n_pad - n_nodes, d), x_bf.dtype)], axis=0)
        x3 = x_bf.reshape(n_pad // 2, 2, d).transpose(0, 2, 1)
        return lax.bitcast_convert_type(x3, jnp.float32)

    support1 = pack_rows(_matmul_bf16(features, W1, bm))
    p = edge_pass(support1, esl, zeros)
    support2 = pack_rows(
        _combine_relu_matmul_bf16(p[0, :n_nodes], p[1, :n_nodes], b1, W3, bm))
    q = edge_pass(support2, esl, zeros)
    return _combine_bias(q[0, :n_nodes], q[1, :n_nodes], b3, bm)
